# baseline (speedup 1.0000x reference)
import jax
import jax.numpy as jnp
from jax import lax
from jax.experimental import pallas as pl
from jax.experimental.pallas import tpu as pltpu
from jax.experimental.pallas import tpu_sc as plsc

VOCAB = 1000000
D_MODEL = 64
BATCH = 4096
SEQ = 200

NC = 2
NS = 16
NW = NC * NS
BBLK = BATCH // NW
NG = 8
NWR = 4
JH = D_MODEL // 8
LANES = 16
WPAD = 133


def _embed_kernel(xt_hbm, table_hbm, out_hbm, idx_v, gbufs, wbufs, gsems, wsems):
    wid = lax.axis_index("s") * NC + lax.axis_index("c")

    pltpu.sync_copy(xt_hbm.at[:, pl.ds(wid * BBLK, BBLK)], idx_v)

    lanes = lax.iota(jnp.int32, LANES)

    for b in range(NG):
        pltpu.async_copy(table_hbm.at[idx_v.at[b]], gbufs[b], gsems.at[b])

    jh_c = [(k * LANES + lanes) // 8 for k in range(D_MODEL // LANES)]
    jl_c = [(k * LANES + lanes) % 8 for k in range(D_MODEL // LANES)]

    def transpose_chunk(g, w):
        def bbody(b4, carry):
            for db in range(4):
                b = b4 * 4 + db
                bidx = jnp.full((LANES,), b, jnp.int32)
                vecs = [
                    g[b, pl.ds(k * LANES, LANES)]
                    for k in range(D_MODEL // LANES)
                ]
                for k, vec in enumerate(vecs):
                    plsc.store_scatter(w, [jh_c[k], jl_c[k], bidx], vec)
            return carry

        lax.fori_loop(0, BBLK // 4, bbody, 0)

    def group_body(grp, carry):
        for b in range(NG):
            s = grp * NG + b
            wb = b % NWR
            pltpu.make_async_copy(
                table_hbm.at[idx_v.at[s]], gbufs[b], gsems.at[b]
            ).wait()

            @pl.when(s >= NWR)
            def _():
                pltpu.make_async_copy(
                    wbufs[wb].at[:, :, pl.ds(0, BBLK)],
                    out_hbm.at[s - NWR, :, wid],
                    wsems.at[wb],
                ).wait()

            transpose_chunk(gbufs[b], wbufs[wb])
            pltpu.async_copy(
                wbufs[wb].at[:, :, pl.ds(0, BBLK)],
                out_hbm.at[s, :, wid],
                wsems.at[wb],
            )

            nxt = s + NG

            @pl.when(nxt < SEQ)
            def _():
                pltpu.async_copy(
                    table_hbm.at[idx_v.at[nxt]], gbufs[b], gsems.at[b]
                )

        return carry

    lax.fori_loop(0, SEQ // NG, group_body, 0)

    for wb in range(NWR):
        s = SEQ - NWR + wb
        pltpu.make_async_copy(
            wbufs[wb].at[:, :, pl.ds(0, BBLK)], out_hbm.at[s, :, wid], wsems.at[wb]
        ).wait()


@jax.jit
def kernel(x, table):
    xt = x.T.astype(jnp.int32)
    mesh = plsc.VectorSubcoreMesh(core_axis_name="c", subcore_axis_name="s")
    run = pl.kernel(
        _embed_kernel,
        out_type=jax.ShapeDtypeStruct((SEQ, JH, NW, 8, BBLK), jnp.float32),
        mesh=mesh,
        scratch_types=[
            pltpu.VMEM((SEQ, BBLK), jnp.int32),
            [pltpu.VMEM((BBLK, D_MODEL), jnp.float32) for _ in range(NG)],
            [pltpu.VMEM((JH, 8, WPAD), jnp.float32) for _ in range(NWR)],
            pltpu.SemaphoreType.DMA((NG,)),
            pltpu.SemaphoreType.DMA((NWR,)),
        ],
        compiler_params=pltpu.CompilerParams(
            use_tc_tiling_on_sc=False, needs_layout_passes=False
        ),
    )
    p = run(xt, table)
    return p.transpose(2, 4, 0, 1, 3).reshape(BATCH, SEQ, D_MODEL)
